# baseline (speedup 1.0000x reference)
import functools

import jax
import jax.numpy as jnp
from jax import lax
from jax.experimental import pallas as pl
from jax.experimental.pallas import tpu as pltpu
from jax.experimental.pallas import tpu_sc as plsc

NUM_USERS = 1000000
EMBED_DIM = 64
BATCH = 16384

NC = 2
NS = 16
L = 16
NW = NC * NS
B_PER_W = BATCH // NW
PACK = 128 // EMBED_DIM
ROWS_PK = NUM_USERS // PACK
CHUNK = 128
HALF = B_PER_W // 2


def _body(uid_hbm, pid_hbm, uemb_hbm, pemb_hbm, out_hbm,
          uidx_v, pidx_v, uidx2_v, pidx2_v, urows_v, prows_v, out_v, sem):
    wid = lax.axis_index("s") * NC + lax.axis_index("c")
    base = wid * B_PER_W

    pltpu.sync_copy(uid_hbm.at[pl.ds(base, B_PER_W)], uidx_v)
    pltpu.sync_copy(pid_hbm.at[pl.ds(base, B_PER_W)], pidx_v)

    def shift(j, _):
        sl = pl.ds(j * L, L)
        uidx2_v[sl] = lax.shift_right_logical(uidx_v[sl], 1)
        pidx2_v[sl] = lax.shift_right_logical(pidx_v[sl], 1)
        return _

    lax.fori_loop(0, B_PER_W // L, shift, None)

    for h in range(2):
        copies = []
        for c in range(HALF // CHUNK):
            s_idx = pl.ds(h * HALF + c * CHUNK, CHUNK)
            s_dst = pl.ds(c * CHUNK, CHUNK)
            copies.append(pltpu.async_copy(
                uemb_hbm.at[uidx2_v.at[s_idx]], urows_v.at[s_dst, :], sem))
            copies.append(pltpu.async_copy(
                pemb_hbm.at[pidx2_v.at[s_idx]], prows_v.at[s_dst, :], sem))
        for cp in copies:
            cp.wait()

        def group(g, _):
            row = g * L + lax.iota(jnp.int32, L)
            upar = (uidx_v[pl.ds(h * HALF + g * L, L)] & 1) * EMBED_DIM
            ppar = (pidx_v[pl.ds(h * HALF + g * L, L)] & 1) * EMBED_DIM
            acc = jnp.zeros((L,), jnp.float32)
            for d in range(EMBED_DIM):
                u = plsc.load_gather(urows_v, [row, upar + d])
                p = plsc.load_gather(prows_v, [row, ppar + d])
                acc = acc + u * p
            out_v[pl.ds(h * HALF + g * L, L)] = 1.0 / (1.0 + jnp.exp(-acc))
            return _

        lax.fori_loop(0, HALF // L, group, None)

    pltpu.sync_copy(out_v, out_hbm.at[pl.ds(base, B_PER_W)])


@functools.partial(jax.jit, static_argnames=())
def _run(user_ids, partner_ids, uemb2, pemb2):
    mesh = plsc.VectorSubcoreMesh(core_axis_name="c", subcore_axis_name="s")
    return pl.kernel(
        _body,
        out_type=jax.ShapeDtypeStruct((BATCH,), jnp.float32),
        mesh=mesh,
        scratch_types=[
            pltpu.VMEM((B_PER_W,), jnp.int32),
            pltpu.VMEM((B_PER_W,), jnp.int32),
            pltpu.VMEM((B_PER_W,), jnp.int32),
            pltpu.VMEM((B_PER_W,), jnp.int32),
            pltpu.VMEM((HALF, PACK * EMBED_DIM), jnp.float32),
            pltpu.VMEM((HALF, PACK * EMBED_DIM), jnp.float32),
            pltpu.VMEM((B_PER_W,), jnp.float32),
            pltpu.SemaphoreType.DMA,
        ],
        compiler_params=pltpu.CompilerParams(needs_layout_passes=False),
    )(user_ids, partner_ids, uemb2, pemb2)


def kernel(user_ids, partner_ids, user_embed, partner_embed):
    uemb2 = user_embed.reshape(ROWS_PK, PACK * EMBED_DIM)
    pemb2 = partner_embed.reshape(ROWS_PK, PACK * EMBED_DIM)
    return _run(user_ids.astype(jnp.int32), partner_ids.astype(jnp.int32),
                uemb2, pemb2)
